# baseline (speedup 1.0000x reference)
import functools

import jax
import jax.numpy as jnp
from jax import lax
from jax.experimental import pallas as pl
from jax.experimental.pallas import tpu as pltpu
from jax.experimental.pallas import tpu_sc as plsc

C = 64
L = 100000
X_H = 512
X_W = 512
IMG_H = 128
IMG_W = 2048

N_TILES = 16
P = 6272
LP = N_TILES * P
ROWS = P // 16
PLANE = X_H * X_W
R_PER_TILE = X_H // N_TILES
CH_PER_CORE = C // 2


def _sc_kernel(x_hbm, img_hbm, px_hbm, py_hbm, ix_hbm, iy_hbm, out_hbm,
               ta, tb, pidx_v, didx_v, vals0, vals1, plane0, plane1,
               gsem0, gsem1, ssem, psem, osem):
    cid = lax.axis_index("c")
    sid = lax.axis_index("s")

    pltpu.sync_copy(px_hbm.at[sid], ta)
    pltpu.sync_copy(py_hbm.at[sid], tb)

    def body_p(j, carry):
        s = pl.ds(j * 16, 16)
        pidx_v[s] = tb[s] * IMG_W + ta[s]
        return carry

    lax.fori_loop(0, ROWS, body_p, 0)

    pltpu.sync_copy(ix_hbm.at[sid], ta)
    pltpu.sync_copy(iy_hbm.at[sid], tb)

    def body_d(j, carry):
        s = pl.ds(j * 16, 16)
        didx_v[s] = ta[s] * X_W + tb[s]
        return carry

    lax.fori_loop(0, ROWS, body_d, 0)

    row0 = sid * R_PER_TILE

    def chan(k):
        return cid * CH_PER_CORE + k

    def gather_src(k):
        return img_hbm.at[pl.ds(chan(k) * PLANE, PLANE)].at[pidx_v]

    def rows_in(k, plane, sem):
        def body(r, carry):
            pltpu.async_copy(x_hbm.at[chan(k), row0 + r, :],
                             plane.at[pl.ds((row0 + r) * X_W, X_W)], sem)
            return carry
        lax.fori_loop(0, R_PER_TILE, body, 0)

    def rows_in_wait(k, plane, sem):
        def body(r, carry):
            pltpu.make_async_copy(x_hbm.at[chan(k), row0 + r, :],
                                  plane.at[pl.ds((row0 + r) * X_W, X_W)],
                                  sem).wait()
            return carry
        lax.fori_loop(0, R_PER_TILE, body, 0)

    def rows_out(k, plane):
        def body(r, carry):
            pltpu.async_copy(plane.at[pl.ds((row0 + r) * X_W, X_W)],
                             out_hbm.at[chan(k), row0 + r, :], osem)
            return carry
        lax.fori_loop(0, R_PER_TILE, body, 0)

    def rows_out_wait(k, plane):
        def body(r, carry):
            pltpu.make_async_copy(plane.at[pl.ds((row0 + r) * X_W, X_W)],
                                  out_hbm.at[chan(k), row0 + r, :],
                                  osem).wait()
            return carry
        lax.fori_loop(0, R_PER_TILE, body, 0)

    pltpu.async_copy(gather_src(0), vals0, gsem0)
    pltpu.async_copy(gather_src(1), vals1, gsem1)
    rows_in(0, plane0, psem)
    rows_in_wait(0, plane0, psem)
    plsc.subcore_barrier()

    def chan_body(k, carry):
        def run(b_static):
            vals = vals0 if b_static == 0 else vals1
            gsem = gsem0 if b_static == 0 else gsem1
            plane = plane0 if b_static == 0 else plane1
            planen = plane1 if b_static == 0 else plane0

            pltpu.make_async_copy(gather_src(k), vals, gsem).wait()
            pltpu.async_copy(vals, plane.at[didx_v], ssem, add=True)

            @pl.when(k >= 1)
            def _writeback_prev():
                rows_out(k - 1, planen)
                rows_out_wait(k - 1, planen)

            @pl.when(k < CH_PER_CORE - 1)
            def _stage_next():
                rows_in(k + 1, planen, psem)
                rows_in_wait(k + 1, planen, psem)

            pltpu.make_async_copy(vals, plane.at[didx_v], ssem).wait()

            @pl.when(k < CH_PER_CORE - 2)
            def _prefetch_gather():
                pltpu.async_copy(gather_src(k + 2), vals, gsem)

            plsc.subcore_barrier()

        @pl.when(k % 2 == 0)
        def _():
            run(0)

        @pl.when(k % 2 == 1)
        def _():
            run(1)

        return carry

    lax.fori_loop(0, CH_PER_CORE, chan_body, 0)

    b_last = (CH_PER_CORE - 1) % 2
    last_plane = plane1 if b_last else plane0
    rows_out(CH_PER_CORE - 1, last_plane)
    rows_out_wait(CH_PER_CORE - 1, last_plane)


def kernel(x, img, index_x, index_y, proj_x, proj_y):
    pad = LP - L
    px = jnp.concatenate([proj_x, jnp.zeros((pad,), jnp.int32)])
    py = jnp.concatenate([proj_y, jnp.zeros((pad,), jnp.int32)])
    ix = jnp.concatenate([index_x.reshape(-1), jnp.full((pad,), X_H, jnp.int32)])
    iy = jnp.concatenate([index_y.reshape(-1), jnp.zeros((pad,), jnp.int32)])
    px = px.reshape(N_TILES, P)
    py = py.reshape(N_TILES, P)
    ix = ix.reshape(N_TILES, P)
    iy = iy.reshape(N_TILES, P)

    x3 = x.reshape(C, X_H, X_W)
    img_flat = img.reshape(-1)

    mesh = plsc.VectorSubcoreMesh(core_axis_name="c", subcore_axis_name="s")
    kern = functools.partial(
        pl.kernel,
        out_type=jax.ShapeDtypeStruct((C, X_H, X_W), jnp.float32),
        mesh=mesh,
        scratch_types=[
            pltpu.VMEM((P,), jnp.int32),
            pltpu.VMEM((P,), jnp.int32),
            pltpu.VMEM((P,), jnp.int32),
            pltpu.VMEM((P,), jnp.int32),
            pltpu.VMEM((P,), jnp.float32),
            pltpu.VMEM((P,), jnp.float32),
            pltpu.VMEM_SHARED((PLANE + 16,), jnp.float32),
            pltpu.VMEM_SHARED((PLANE + 16,), jnp.float32),
            pltpu.SemaphoreType.DMA,
            pltpu.SemaphoreType.DMA,
            pltpu.SemaphoreType.DMA,
            pltpu.SemaphoreType.DMA,
            pltpu.SemaphoreType.DMA,
        ],
    )(_sc_kernel)

    return kern(x3, img_flat, px, py, ix, iy)
